# Initial kernel scaffold; baseline (speedup 1.0000x reference)
import jax
import jax.numpy as jnp
from jax.experimental import pallas as pl


def kernel(nodes, features, neigh_r1, neigh_r2, neigh_r3, alpha):
    raise NotImplementedError("write your pallas kernel here")


# speedup vs baseline: 2.2550x; 2.2550x over previous
import jax
import jax.numpy as jnp
from jax import lax
from jax.experimental import pallas as pl
from jax.experimental.pallas import tpu as pltpu
from jax.experimental.pallas import tpu_sc as plsc

L = 16
NC, NS = 2, 16
NW = NC * NS
EMBED2 = 256
DEG = 64
NREL = 3
CHUNKS = EMBED2 // L


def _kernel_body(nodes_hbm, feat_hbm, n1_hbm, n2_hbm, n3_hbm, alphat_hbm,
                 out_hbm, nodes_v, n1_v, n2_v, n3_v, alphat_v, w1_v,
                 g0, g1, g2, seed_v, stage_v, s0, s1, s2, s_seed):
    npw = nodes_v.shape[0]
    wid = lax.axis_index("s") * NC + lax.axis_index("c")
    base = wid * npw
    gbufs = (g0, g1, g2)
    sems = (s0, s1, s2)
    slabs = (n1_v, n2_v, n3_v)

    pltpu.sync_copy(nodes_hbm.at[pl.ds(base, npw)], nodes_v)
    pltpu.sync_copy(n1_hbm.at[pl.ds(base, npw)], n1_v)
    pltpu.sync_copy(n2_hbm.at[pl.ds(base, npw)], n2_v)
    pltpu.sync_copy(n3_hbm.at[pl.ds(base, npw)], n3_v)
    pltpu.sync_copy(alphat_hbm, alphat_v)

    pltpu.async_copy(feat_hbm.at[nodes_v], seed_v, s_seed)

    def fire(i, r):
        pltpu.async_copy(feat_hbm.at[slabs[r].at[i]], gbufs[r], sems[r])

    for r in range(NREL):
        fire(0, r)

    inv_deg = jnp.float32(1.0 / DEG)
    for c in range(CHUNKS):
        a0 = alphat_v[0, pl.ds(c * L, L)]
        a1 = alphat_v[1, pl.ds(c * L, L)]
        a2 = alphat_v[2, pl.ds(c * L, L)]
        m = jnp.maximum(jnp.maximum(a0, a1), a2)
        e0 = jnp.exp(a0 - m)
        e1 = jnp.exp(a1 - m)
        e2 = jnp.exp(a2 - m)
        w1_v[pl.ds(c * L, L)] = (e1 / (e0 + e1 + e2)) * inv_deg

    pltpu.make_async_copy(feat_hbm.at[nodes_v], seed_v, s_seed).wait()

    def node_body(i, carry):
        accs = [jnp.zeros((L,), jnp.float32) for _ in range(CHUNKS)]
        for r in range(NREL):
            pltpu.make_async_copy(feat_hbm.at[slabs[r].at[i]], gbufs[r],
                                  sems[r]).wait()
            buf = gbufs[r]

            def red_body(t, acc, buf=buf):
                acc = list(acc)
                for rr in range(8):
                    row = t * 8 + rr
                    for c in range(CHUNKS):
                        acc[c] = acc[c] + buf[row, pl.ds(c * L, L)]
                return tuple(acc)

            accs = list(lax.fori_loop(0, DEG // 8, red_body, tuple(accs)))

            @pl.when(i + 1 < npw)
            def _():
                fire(i + 1, r)

        for c in range(CHUNKS):
            stage_v[i, pl.ds(c * L, L)] = seed_v[i, pl.ds(c * L, L)]
            stage_v[i, pl.ds(EMBED2 + c * L, L)] = (
                accs[c] * w1_v[pl.ds(c * L, L)])
        return carry

    lax.fori_loop(0, npw, node_body, 0)

    pltpu.sync_copy(stage_v, out_hbm.at[pl.ds(base, npw)])


def kernel(nodes, features, neigh_r1, neigh_r2, neigh_r3, alpha):
    batch = nodes.shape[0]
    npw = batch // NW
    alphat = jnp.transpose(alpha)

    k = pl.kernel(
        _kernel_body,
        out_type=jax.ShapeDtypeStruct((batch, 2 * EMBED2), jnp.float32),
        mesh=plsc.VectorSubcoreMesh(core_axis_name="c", subcore_axis_name="s"),
        scratch_types=[
            pltpu.VMEM((npw,), jnp.int32),
            pltpu.VMEM((npw, DEG), jnp.int32),
            pltpu.VMEM((npw, DEG), jnp.int32),
            pltpu.VMEM((npw, DEG), jnp.int32),
            pltpu.VMEM((NREL, EMBED2), jnp.float32),
            pltpu.VMEM((EMBED2,), jnp.float32),
            pltpu.VMEM((DEG, EMBED2), jnp.float32),
            pltpu.VMEM((DEG, EMBED2), jnp.float32),
            pltpu.VMEM((DEG, EMBED2), jnp.float32),
            pltpu.VMEM((npw, EMBED2), jnp.float32),
            pltpu.VMEM((npw, 2 * EMBED2), jnp.float32),
            pltpu.SemaphoreType.DMA,
            pltpu.SemaphoreType.DMA,
            pltpu.SemaphoreType.DMA,
            pltpu.SemaphoreType.DMA,
        ],
    )
    return k(nodes, features, neigh_r1, neigh_r2, neigh_r3, alphat)
